# baseline (speedup 1.0000x reference)
import functools

import jax
import jax.numpy as jnp
from jax import lax
from jax.experimental import pallas as pl
from jax.experimental.pallas import tpu as pltpu
from jax.experimental.pallas import tpu_sc as plsc

_H = 512
_LANES = 16
_NCORES = 2
_NSUB = 16
_NW = _NCORES * _NSUB
_CH = _H // _LANES
_C = 16


def _tc_body(f0, f1, f2, f3, f4, f5, f6, f7, f8, w_ref, out_ref):
    w = w_ref[0, :][None, :]
    feats = [r[...] for r in (f0, f1, f2, f3, f4, f5, f6, f7, f8)]
    scores = [jnp.sum(f * w, axis=1, keepdims=True) for f in feats]
    m = scores[0]
    for s in scores[1:]:
        m = jnp.maximum(m, s)
    exps = [jnp.exp(s - m) for s in scores]
    denom = exps[0]
    for e in exps[1:]:
        denom = denom + e
    inv = 1.0 / denom
    acc = feats[0] * (exps[0] * inv)
    for i in range(1, 9):
        acc = acc + feats[i] * (exps[i] * inv)
    out_ref[...] = acc


def _tc_fuse(feats, w2, n_rows, blk):
    feat_spec = pl.BlockSpec((blk, _H), lambda i: (i, 0))
    return pl.pallas_call(
        _tc_body,
        grid=(n_rows // blk,),
        in_specs=[feat_spec] * 9 + [pl.BlockSpec((1, _H), lambda i: (0, 0))],
        out_specs=feat_spec,
        out_shape=jax.ShapeDtypeStruct((n_rows, _H), jnp.float32),
    )(*feats, w2)


def _make_sc(n_rows_sc, row0):
    C = 8
    rpw = n_rows_sc // _NW
    nch = rpw // C
    mesh = plsc.VectorSubcoreMesh(core_axis_name="c", subcore_axis_name="s")
    scratch = ([pltpu.VMEM((C, _H), jnp.float32) for _ in range(18)]
               + [pltpu.VMEM((_H,), jnp.float32),
                  pltpu.VMEM((C, _H), jnp.float32),
                  pltpu.VMEM((C, _H), jnp.float32),
                  pltpu.SemaphoreType.DMA,
                  pltpu.SemaphoreType.DMA,
                  pltpu.SemaphoreType.DMA,
                  pltpu.SemaphoreType.DMA])

    @functools.partial(
        pl.kernel, mesh=mesh,
        out_type=jax.ShapeDtypeStruct((n_rows_sc, _H), jnp.float32),
        scratch_types=scratch,
        compiler_params=pltpu.CompilerParams(needs_layout_passes=False),
    )
    def sc_kernel(f0, f1, f2, f3, f4, f5, f6, f7, f8, w_hbm, out_hbm, *sc):
        fhs = (f0, f1, f2, f3, f4, f5, f6, f7, f8)
        buf_a, buf_b = sc[0:9], sc[9:18]
        wv, ov_a, ov_b, s_ia, s_ib, s_oa, s_ob = sc[18:]
        wid = lax.axis_index("s") * _NCORES + lax.axis_index("c")
        pltpu.sync_copy(w_hbm, wv)
        base0 = row0 + wid * rpw
        last_base = row0 + n_rows_sc - C

        def start_in(base, bufs, sem):
            for i in range(9):
                pltpu.async_copy(fhs[i].at[pl.ds(base, C), :], bufs[i], sem)

        def wait_in(bufs, sem):
            for i in range(9):
                pltpu.make_async_copy(
                    fhs[i].at[pl.ds(0, C), :], bufs[i], sem).wait()

        def start_out(ov, base, sem):
            pltpu.async_copy(ov, out_hbm.at[pl.ds(base - row0, C), :], sem)

        def wait_out(ov, sem):
            pltpu.make_async_copy(
                ov, out_hbm.at[pl.ds(0, C), :], sem).wait()

        def compute_chunk(bufs, ov):
            def row_group(rg, c2):
                r0 = 2 * rg
                rows = (r0, r0 + 1)
                accs = {}
                for k in range(_CH):
                    wk = wv[pl.ds(_LANES * k, _LANES)]
                    for ri, r in enumerate(rows):
                        for i in range(9):
                            t = bufs[i][r, pl.ds(_LANES * k, _LANES)] * wk
                            accs[(ri, i)] = t if k == 0 else accs[(ri, i)] + t
                for ri, r in enumerate(rows):
                    ss = [jnp.sum(accs[(ri, i)]) for i in range(9)]
                    m = ss[0]
                    for s in ss[1:]:
                        m = jnp.maximum(m, s)
                    es = [jnp.exp(jnp.full((_LANES,), s - m)) for s in ss]
                    den = es[0]
                    for e in es[1:]:
                        den = den + e
                    atts = [e / den for e in es]
                    for k in range(_CH):
                        a0 = bufs[0][r, pl.ds(_LANES * k, _LANES)] * atts[0]
                        a1 = bufs[1][r, pl.ds(_LANES * k, _LANES)] * atts[1]
                        a2 = bufs[2][r, pl.ds(_LANES * k, _LANES)] * atts[2]
                        for i in range(3, 9):
                            t = bufs[i][r, pl.ds(_LANES * k, _LANES)] * atts[i]
                            if i % 3 == 0:
                                a0 = a0 + t
                            elif i % 3 == 1:
                                a1 = a1 + t
                            else:
                                a2 = a2 + t
                        ov[r, pl.ds(_LANES * k, _LANES)] = a0 + (a1 + a2)
                return c2
            lax.fori_loop(0, C // 2, row_group, 0)

        start_in(base0, buf_a, s_ia)

        def super_body(g, carry):
            base_a = base0 + (2 * g) * C
            base_b = base_a + C
            base_a2 = jnp.minimum(base_a + 2 * C, last_base)
            start_in(base_b, buf_b, s_ib)
            wait_in(buf_a, s_ia)

            @pl.when(g > 0)
            def _():
                wait_out(ov_a, s_oa)
            compute_chunk(buf_a, ov_a)
            start_out(ov_a, base_a, s_oa)

            start_in(base_a2, buf_a, s_ia)
            wait_in(buf_b, s_ib)

            @pl.when(g > 0)
            def _():
                wait_out(ov_b, s_ob)
            compute_chunk(buf_b, ov_b)
            start_out(ov_b, base_b, s_ob)
            return carry
        lax.fori_loop(0, nch // 2, super_body, 0)

        wait_in(buf_a, s_ia)
        wait_out(ov_a, s_oa)
        wait_out(ov_b, s_ob)

    return sc_kernel


def kernel(feat_0, feat_1, feat_2, feat_3, feat_4, feat_5, feat_6, feat_7,
           feat_8, W):
    S, B, H = feat_0.shape
    R = S * B
    feats = [f.reshape(R, H) for f in (feat_0, feat_1, feat_2, feat_3, feat_4,
                                       feat_5, feat_6, feat_7, feat_8)]
    w2 = W.reshape(1, H)

    n_sc = 4096 if R >= 8192 else 0
    n_tc = R - n_sc

    tc_out = _tc_fuse(feats, w2, n_tc, min(512, n_tc))
    if n_sc == 0:
        return tc_out.reshape(S, B, H)

    sc_out = _make_sc(n_sc, n_tc)(*feats, W)
    full = jnp.concatenate([tc_out, sc_out], axis=0)
    return full.reshape(S, B, H)


# speedup vs baseline: 2.1366x; 2.1366x over previous
import functools

import jax
import jax.numpy as jnp
from jax import lax
from jax.experimental import pallas as pl
from jax.experimental.pallas import tpu as pltpu
from jax.experimental.pallas import tpu_sc as plsc

_H = 512
_LANES = 16
_NCORES = 2
_NSUB = 16
_NW = _NCORES * _NSUB
_CH = _H // _LANES
_C = 16


def _tc_body(f0, f1, f2, f3, f4, f5, f6, f7, f8, w_ref, out_ref):
    w = w_ref[0, :][None, :]
    feats = [r[...] for r in (f0, f1, f2, f3, f4, f5, f6, f7, f8)]
    scores = [jnp.sum(f * w, axis=1, keepdims=True) for f in feats]
    m = scores[0]
    for s in scores[1:]:
        m = jnp.maximum(m, s)
    exps = [jnp.exp(s - m) for s in scores]
    denom = exps[0]
    for e in exps[1:]:
        denom = denom + e
    inv = 1.0 / denom
    acc = feats[0] * (exps[0] * inv)
    for i in range(1, 9):
        acc = acc + feats[i] * (exps[i] * inv)
    out_ref[...] = acc


def _tc_fuse(feats, w2, n_rows, blk):
    feat_spec = pl.BlockSpec((blk, _H), lambda i: (i, 0))
    return pl.pallas_call(
        _tc_body,
        grid=(n_rows // blk,),
        in_specs=[feat_spec] * 9 + [pl.BlockSpec((1, _H), lambda i: (0, 0))],
        out_specs=feat_spec,
        out_shape=jax.ShapeDtypeStruct((n_rows, _H), jnp.float32),
    )(*feats, w2)


def _make_sc(n_rows_sc, row0):
    C = 8
    rpw = n_rows_sc // _NW
    nch = rpw // C
    mesh = plsc.VectorSubcoreMesh(core_axis_name="c", subcore_axis_name="s")
    scratch = ([pltpu.VMEM((C, _H), jnp.float32) for _ in range(18)]
               + [pltpu.VMEM((_H,), jnp.float32),
                  pltpu.VMEM((C, _H), jnp.float32),
                  pltpu.VMEM((C, _H), jnp.float32),
                  pltpu.SemaphoreType.DMA,
                  pltpu.SemaphoreType.DMA,
                  pltpu.SemaphoreType.DMA,
                  pltpu.SemaphoreType.DMA])

    @functools.partial(
        pl.kernel, mesh=mesh,
        out_type=jax.ShapeDtypeStruct((n_rows_sc, _H), jnp.float32),
        scratch_types=scratch,
        compiler_params=pltpu.CompilerParams(needs_layout_passes=False),
    )
    def sc_kernel(f0, f1, f2, f3, f4, f5, f6, f7, f8, w_hbm, out_hbm, *sc):
        fhs = (f0, f1, f2, f3, f4, f5, f6, f7, f8)
        buf_a, buf_b = sc[0:9], sc[9:18]
        wv, ov_a, ov_b, s_ia, s_ib, s_oa, s_ob = sc[18:]
        wid = lax.axis_index("s") * _NCORES + lax.axis_index("c")
        pltpu.sync_copy(w_hbm, wv)
        base0 = row0 + wid * rpw
        last_base = row0 + n_rows_sc - C

        def start_in(base, bufs, sem):
            for i in range(9):
                pltpu.async_copy(fhs[i].at[pl.ds(base, C), :], bufs[i], sem)

        def wait_in(bufs, sem):
            for i in range(9):
                pltpu.make_async_copy(
                    fhs[i].at[pl.ds(0, C), :], bufs[i], sem).wait()

        def start_out(ov, base, sem):
            pltpu.async_copy(ov, out_hbm.at[pl.ds(base - row0, C), :], sem)

        def wait_out(ov, sem):
            pltpu.make_async_copy(
                ov, out_hbm.at[pl.ds(0, C), :], sem).wait()

        def compute_chunk(bufs, ov):
            def row_body(r, c2):
                def score_k(kk, accs):
                    res = list(accs)
                    for u in range(4):
                        k = (kk * 4 + u) * _LANES
                        wk = wv[pl.ds(k, _LANES)]
                        for i in range(9):
                            res[i] = res[i] + bufs[i][r, pl.ds(k, _LANES)] * wk
                    return tuple(res)
                zero = jnp.zeros((_LANES,), jnp.float32)
                accs = lax.fori_loop(0, _CH // 4, score_k, (zero,) * 9)
                ss = [jnp.sum(a) for a in accs]
                m = ss[0]
                for s in ss[1:]:
                    m = jnp.maximum(m, s)
                es = [jnp.exp(jnp.full((_LANES,), s - m)) for s in ss]
                den = es[0]
                for e in es[1:]:
                    den = den + e
                atts = [e / den for e in es]

                def weight_k(kk, c3):
                    for u in range(4):
                        k = (kk * 4 + u) * _LANES
                        a0 = bufs[0][r, pl.ds(k, _LANES)] * atts[0]
                        a1 = bufs[1][r, pl.ds(k, _LANES)] * atts[1]
                        a2 = bufs[2][r, pl.ds(k, _LANES)] * atts[2]
                        for i in range(3, 9):
                            t = bufs[i][r, pl.ds(k, _LANES)] * atts[i]
                            if i % 3 == 0:
                                a0 = a0 + t
                            elif i % 3 == 1:
                                a1 = a1 + t
                            else:
                                a2 = a2 + t
                        ov[r, pl.ds(k, _LANES)] = a0 + (a1 + a2)
                    return c3
                lax.fori_loop(0, _CH // 4, weight_k, 0)
                return c2
            lax.fori_loop(0, C, row_body, 0)

        start_in(base0, buf_a, s_ia)

        def super_body(g, carry):
            base_a = base0 + (2 * g) * C
            base_b = base_a + C
            base_a2 = jnp.minimum(base_a + 2 * C, last_base)
            start_in(base_b, buf_b, s_ib)
            wait_in(buf_a, s_ia)

            @pl.when(g > 0)
            def _():
                wait_out(ov_a, s_oa)
            compute_chunk(buf_a, ov_a)
            start_out(ov_a, base_a, s_oa)

            start_in(base_a2, buf_a, s_ia)
            wait_in(buf_b, s_ib)

            @pl.when(g > 0)
            def _():
                wait_out(ov_b, s_ob)
            compute_chunk(buf_b, ov_b)
            start_out(ov_b, base_b, s_ob)
            return carry
        lax.fori_loop(0, nch // 2, super_body, 0)

        wait_in(buf_a, s_ia)
        wait_out(ov_a, s_oa)
        wait_out(ov_b, s_ob)

    return sc_kernel


def kernel(feat_0, feat_1, feat_2, feat_3, feat_4, feat_5, feat_6, feat_7,
           feat_8, W):
    S, B, H = feat_0.shape
    R = S * B
    feats = [f.reshape(R, H) for f in (feat_0, feat_1, feat_2, feat_3, feat_4,
                                       feat_5, feat_6, feat_7, feat_8)]
    w2 = W.reshape(1, H)

    n_sc = 4096 if R >= 8192 else 0
    n_tc = R - n_sc

    tc_out = _tc_fuse(feats, w2, n_tc, min(512, n_tc))
    if n_sc == 0:
        return tc_out.reshape(S, B, H)

    sc_out = _make_sc(n_sc, n_tc)(*feats, W)
    full = jnp.concatenate([tc_out, sc_out], axis=0)
    return full.reshape(S, B, H)


# speedup vs baseline: 2.3969x; 1.1218x over previous
import functools

import jax
import jax.numpy as jnp
from jax import lax
from jax.experimental import pallas as pl
from jax.experimental.pallas import tpu as pltpu
from jax.experimental.pallas import tpu_sc as plsc

_H = 512
_LANES = 16
_NCORES = 2
_NSUB = 16
_NW = _NCORES * _NSUB
_CH = _H // _LANES
_C = 16


def _tc_body(f0, f1, f2, f3, f4, f5, f6, f7, f8, w_ref, out_ref):
    w = w_ref[0, :][None, :]
    feats = [r[...] for r in (f0, f1, f2, f3, f4, f5, f6, f7, f8)]
    scores = [jnp.sum(f * w, axis=1, keepdims=True) for f in feats]
    m = scores[0]
    for s in scores[1:]:
        m = jnp.maximum(m, s)
    exps = [jnp.exp(s - m) for s in scores]
    denom = exps[0]
    for e in exps[1:]:
        denom = denom + e
    inv = 1.0 / denom
    acc = feats[0] * (exps[0] * inv)
    for i in range(1, 9):
        acc = acc + feats[i] * (exps[i] * inv)
    out_ref[...] = acc


def _tc_fuse(feats, w2, n_rows, blk, out_rows):
    feat_spec = pl.BlockSpec((blk, _H), lambda i: (i, 0))
    return pl.pallas_call(
        _tc_body,
        grid=(n_rows // blk,),
        in_specs=[feat_spec] * 9 + [pl.BlockSpec((1, _H), lambda i: (0, 0))],
        out_specs=feat_spec,
        out_shape=jax.ShapeDtypeStruct((out_rows, _H), jnp.float32),
    )(*feats, w2)


def _make_sc(n_rows_sc, row0):
    C = 8
    rpw = n_rows_sc // _NW
    nch = rpw // C
    mesh = plsc.VectorSubcoreMesh(core_axis_name="c", subcore_axis_name="s")
    scratch = ([pltpu.VMEM((C, _H), jnp.float32) for _ in range(18)]
               + [pltpu.VMEM((_H,), jnp.float32),
                  pltpu.VMEM((C, _H), jnp.float32),
                  pltpu.VMEM((C, _H), jnp.float32),
                  pltpu.SemaphoreType.DMA,
                  pltpu.SemaphoreType.DMA,
                  pltpu.SemaphoreType.DMA,
                  pltpu.SemaphoreType.DMA])

    @functools.partial(
        pl.kernel, mesh=mesh,
        out_type=jax.ShapeDtypeStruct((n_rows_sc, _H), jnp.float32),
        scratch_types=scratch,
        compiler_params=pltpu.CompilerParams(needs_layout_passes=False),
    )
    def sc_kernel(f0, f1, f2, f3, f4, f5, f6, f7, f8, w_hbm, out_hbm, *sc):
        fhs = (f0, f1, f2, f3, f4, f5, f6, f7, f8)
        buf_a, buf_b = sc[0:9], sc[9:18]
        wv, ov_a, ov_b, s_ia, s_ib, s_oa, s_ob = sc[18:]
        wid = lax.axis_index("s") * _NCORES + lax.axis_index("c")
        pltpu.sync_copy(w_hbm, wv)
        base0 = row0 + wid * rpw
        last_base = row0 + n_rows_sc - C

        def start_in(base, bufs, sem):
            for i in range(9):
                pltpu.async_copy(fhs[i].at[pl.ds(base, C), :], bufs[i], sem)

        def wait_in(bufs, sem):
            for i in range(9):
                pltpu.make_async_copy(
                    fhs[i].at[pl.ds(0, C), :], bufs[i], sem).wait()

        def start_out(ov, base, sem):
            pltpu.async_copy(ov, out_hbm.at[pl.ds(base - row0, C), :], sem)

        def wait_out(ov, sem):
            pltpu.make_async_copy(
                ov, out_hbm.at[pl.ds(0, C), :], sem).wait()

        def compute_chunk(bufs, ov):
            def row_body(r, c2):
                def score_k(kk, accs):
                    res = list(accs)
                    for u in range(8):
                        k = (kk * 8 + u) * _LANES
                        wk = wv[pl.ds(k, _LANES)]
                        for i in range(9):
                            res[i] = res[i] + bufs[i][r, pl.ds(k, _LANES)] * wk
                    return tuple(res)
                zero = jnp.zeros((_LANES,), jnp.float32)
                accs = lax.fori_loop(0, _CH // 8, score_k, (zero,) * 9)
                ss = [jnp.sum(a) for a in accs]
                m = ss[0]
                for s in ss[1:]:
                    m = jnp.maximum(m, s)
                es = [jnp.exp(jnp.full((_LANES,), s - m)) for s in ss]
                den = es[0]
                for e in es[1:]:
                    den = den + e
                atts = [e / den for e in es]

                def weight_k(kk, c3):
                    for u in range(8):
                        k = (kk * 8 + u) * _LANES
                        a0 = bufs[0][r, pl.ds(k, _LANES)] * atts[0]
                        a1 = bufs[1][r, pl.ds(k, _LANES)] * atts[1]
                        a2 = bufs[2][r, pl.ds(k, _LANES)] * atts[2]
                        for i in range(3, 9):
                            t = bufs[i][r, pl.ds(k, _LANES)] * atts[i]
                            if i % 3 == 0:
                                a0 = a0 + t
                            elif i % 3 == 1:
                                a1 = a1 + t
                            else:
                                a2 = a2 + t
                        ov[r, pl.ds(k, _LANES)] = a0 + (a1 + a2)
                    return c3
                lax.fori_loop(0, _CH // 8, weight_k, 0)
                return c2
            lax.fori_loop(0, C, row_body, 0)

        start_in(base0, buf_a, s_ia)

        def super_body(g, carry):
            base_a = base0 + (2 * g) * C
            base_b = base_a + C
            base_a2 = jnp.minimum(base_a + 2 * C, last_base)
            start_in(base_b, buf_b, s_ib)
            wait_in(buf_a, s_ia)

            @pl.when(g > 0)
            def _():
                wait_out(ov_a, s_oa)
            compute_chunk(buf_a, ov_a)
            start_out(ov_a, base_a, s_oa)

            start_in(base_a2, buf_a, s_ia)
            wait_in(buf_b, s_ib)

            @pl.when(g > 0)
            def _():
                wait_out(ov_b, s_ob)
            compute_chunk(buf_b, ov_b)
            start_out(ov_b, base_b, s_ob)
            return carry
        lax.fori_loop(0, nch // 2, super_body, 0)

        wait_in(buf_a, s_ia)
        wait_out(ov_a, s_oa)
        wait_out(ov_b, s_ob)

    return sc_kernel


def kernel(feat_0, feat_1, feat_2, feat_3, feat_4, feat_5, feat_6, feat_7,
           feat_8, W):
    S, B, H = feat_0.shape
    R = S * B
    feats = [f.reshape(R, H) for f in (feat_0, feat_1, feat_2, feat_3, feat_4,
                                       feat_5, feat_6, feat_7, feat_8)]
    w2 = W.reshape(1, H)

    n_sc = 3584 if R == 16384 else 0
    n_tc = R - n_sc

    if n_sc == 0:
        tc_out = _tc_fuse(feats, w2, n_tc, min(512, n_tc), n_tc)
        return tc_out.reshape(S, B, H)

    tc_out = _tc_fuse(feats, w2, n_tc, 512, R)
    sc_out = _make_sc(n_sc, n_tc)(*feats, W)
    full = lax.dynamic_update_slice(tc_out, sc_out, (n_tc, 0))
    return full.reshape(S, B, H)


# speedup vs baseline: 2.9939x; 1.2491x over previous
import jax
import jax.numpy as jnp
from jax.experimental import pallas as pl
from jax.experimental.pallas import tpu as pltpu


def _fuse_body(f0, f1, f2, f3, f4, f5, f6, f7, f8, w_ref, out_ref):
    w = w_ref[0, :][None, :]
    feats = [r[...] for r in (f0, f1, f2, f3, f4, f5, f6, f7, f8)]
    scores = [jnp.sum(f * w, axis=1, keepdims=True) for f in feats]
    m = scores[0]
    for s in scores[1:]:
        m = jnp.maximum(m, s)
    exps = [jnp.exp(s - m) for s in scores]
    denom = exps[0]
    for e in exps[1:]:
        denom = denom + e
    inv = 1.0 / denom
    acc = feats[0] * (exps[0] * inv)
    for i in range(1, 9):
        acc = acc + feats[i] * (exps[i] * inv)
    out_ref[...] = acc


def kernel(feat_0, feat_1, feat_2, feat_3, feat_4, feat_5, feat_6, feat_7,
           feat_8, W):
    S, B, H = feat_0.shape
    R = S * B
    blk = min(512, R)
    feats = [f.reshape(R, H) for f in (feat_0, feat_1, feat_2, feat_3, feat_4,
                                       feat_5, feat_6, feat_7, feat_8)]
    w2 = W.reshape(1, H)
    feat_spec = pl.BlockSpec((blk, H), lambda i: (i, 0))
    out = pl.pallas_call(
        _fuse_body,
        grid=(R // blk,),
        in_specs=[feat_spec] * 9 + [pl.BlockSpec((1, H), lambda i: (0, 0))],
        out_specs=feat_spec,
        out_shape=jax.ShapeDtypeStruct((R, H), feat_0.dtype),
        compiler_params=pltpu.CompilerParams(
            dimension_semantics=("arbitrary",)),
    )(*feats, w2)
    return out.reshape(S, B, H)
